# Initial kernel scaffold; baseline (speedup 1.0000x reference)
import jax
import jax.numpy as jnp
from jax.experimental import pallas as pl


def kernel(input, indx_seqs, weight, bias):
    raise NotImplementedError("write your pallas kernel here")


# speedup vs baseline: 5.5885x; 5.5885x over previous
import functools

import jax
import jax.numpy as jnp
from jax import lax
from jax.experimental import pallas as pl
from jax.experimental.pallas import tpu as pltpu
from jax.experimental.pallas import tpu_sc as plsc

B, D = 2048, 2048
O, F = 2048, 32

NC, NS = 2, 16
NW = NC * NS
OBLK = 32
NBLK = O // OBLK
BLK_PER_W = NBLK // NW
_ZUNROLL = 8


def _sc_scatter_body(idx_hbm, w_hbm, out_hbm, blk, idxs, ws):
    wid = lax.axis_index("s") * NC + lax.axis_index("c")
    zeros16 = jnp.zeros((16,), jnp.float32)

    for rep in range(BLK_PER_W):
        b = wid * BLK_PER_W + rep
        o0 = b * OBLK

        def zero_body(i, carry):
            c = i * 16 * _ZUNROLL
            for u in range(_ZUNROLL):
                blk[pl.ds(c + u * 16, 16)] = zeros16
            return carry

        lax.fori_loop(0, (OBLK * D) // (16 * _ZUNROLL), zero_body, 0)

        pltpu.sync_copy(idx_hbm.at[pl.ds(o0, OBLK)], idxs)
        pltpu.sync_copy(w_hbm.at[pl.ds(o0, OBLK)], ws)

        for ol in range(OBLK):
            base = jnp.full((16,), ol * D, jnp.int32)
            for h in range(F // 16):
                iv = idxs[ol, pl.ds(h * 16, 16)]
                wv = ws[ol, pl.ds(h * 16, 16)]
                plsc.addupdate_scatter(blk, [base + iv], wv)

        pltpu.sync_copy(blk, out_hbm.at[pl.ds(o0 * D, OBLK * D)])


@functools.cache
def _sc_scatter():
    return functools.partial(
        pl.kernel,
        out_type=jax.ShapeDtypeStruct((O * D,), jnp.float32),
        mesh=plsc.VectorSubcoreMesh(
            core_axis_name="c", subcore_axis_name="s", num_cores=NC, num_subcores=NS
        ),
        scratch_types=[
            pltpu.VMEM((OBLK * D,), jnp.float32),
            pltpu.VMEM((OBLK, F), jnp.int32),
            pltpu.VMEM((OBLK, F), jnp.float32),
        ],
        compiler_params=pltpu.CompilerParams(
            use_tc_tiling_on_sc=False, needs_layout_passes=False
        ),
    )(_sc_scatter_body)


TB, TO = 256, 256


def _mm_body(x_ref, w_ref, b_ref, o_ref):
    o_ref[...] = (
        lax.dot_general(
            x_ref[...],
            w_ref[...],
            (((1,), (1,)), ((), ())),
            preferred_element_type=jnp.float32,
        )
        + b_ref[...]
    )


_matmul = pl.pallas_call(
    _mm_body,
    grid=(B // TB, O // TO),
    in_specs=[
        pl.BlockSpec((TB, D), lambda i, j: (i, 0)),
        pl.BlockSpec((TO, D), lambda i, j: (j, 0)),
        pl.BlockSpec((1, TO), lambda i, j: (0, j)),
    ],
    out_specs=pl.BlockSpec((TB, TO), lambda i, j: (i, j)),
    out_shape=jax.ShapeDtypeStruct((B, O), jnp.float32),
)


def kernel(input, indx_seqs, weight, bias):
    w_dense = _sc_scatter()(indx_seqs.astype(jnp.int32), weight).reshape(O, D)
    return _matmul(input, w_dense, bias.reshape(1, O))


# speedup vs baseline: 8.9754x; 1.6061x over previous
import functools

import jax
import jax.numpy as jnp
from jax import lax
from jax.experimental import pallas as pl
from jax.experimental.pallas import tpu as pltpu
from jax.experimental.pallas import tpu_sc as plsc

B, D = 2048, 2048
O, F = 2048, 32

NC, NS = 2, 16
NW = NC * NS
OBLK = 32
NBLK = O // OBLK
BLK_PER_W = NBLK // NW
_ZUNROLL = 8


def _sc_scatter_body(idx_hbm, w_hbm, out_hbm, blk, idxs, ws):
    wid = lax.axis_index("s") * NC + lax.axis_index("c")
    zeros16 = jnp.zeros((16,), jnp.float32)

    for rep in range(BLK_PER_W):
        b = wid * BLK_PER_W + rep
        o0 = b * OBLK

        def zero_body(i, carry):
            c = i * 16 * _ZUNROLL
            for u in range(_ZUNROLL):
                blk[pl.ds(c + u * 16, 16)] = zeros16
            return carry

        lax.fori_loop(0, (OBLK * D) // (16 * _ZUNROLL), zero_body, 0)

        pltpu.sync_copy(idx_hbm.at[pl.ds(o0, OBLK)], idxs)
        pltpu.sync_copy(w_hbm.at[pl.ds(o0, OBLK)], ws)

        for ol in range(OBLK):
            base = jnp.full((16,), ol * D, jnp.int32)
            for h in range(F // 16):
                iv = idxs[ol, pl.ds(h * 16, 16)]
                wv = ws[ol, pl.ds(h * 16, 16)]
                plsc.addupdate_scatter(blk, [base + iv], wv)

        pltpu.sync_copy(blk, out_hbm.at[pl.ds(o0 * D, OBLK * D)])


@functools.cache
def _sc_scatter():
    return functools.partial(
        pl.kernel,
        out_type=jax.ShapeDtypeStruct((O * D,), jnp.float32),
        mesh=plsc.VectorSubcoreMesh(
            core_axis_name="c", subcore_axis_name="s", num_cores=NC, num_subcores=NS
        ),
        scratch_types=[
            pltpu.VMEM((OBLK * D,), jnp.float32),
            pltpu.VMEM((OBLK, F), jnp.int32),
            pltpu.VMEM((OBLK, F), jnp.float32),
        ],
        compiler_params=pltpu.CompilerParams(
            use_tc_tiling_on_sc=False, needs_layout_passes=False
        ),
    )(_sc_scatter_body)


TB, TO = 1024, 1024


def _mm_body(x_ref, w_ref, b_ref, o_ref):
    o_ref[...] = (
        lax.dot_general(
            x_ref[...],
            w_ref[...],
            (((1,), (1,)), ((), ())),
            preferred_element_type=jnp.float32,
        )
        + b_ref[...]
    )


_matmul = pl.pallas_call(
    _mm_body,
    grid=(B // TB, O // TO),
    in_specs=[
        pl.BlockSpec((TB, D), lambda i, j: (i, 0)),
        pl.BlockSpec((TO, D), lambda i, j: (j, 0)),
        pl.BlockSpec((1, TO), lambda i, j: (0, j)),
    ],
    out_specs=pl.BlockSpec((TB, TO), lambda i, j: (i, j)),
    out_shape=jax.ShapeDtypeStruct((B, O), jnp.float32),
)


def kernel(input, indx_seqs, weight, bias):
    w_dense = _sc_scatter()(indx_seqs.astype(jnp.int32), weight).reshape(O, D)
    return _matmul(input, w_dense, bias.reshape(1, O))


# speedup vs baseline: 9.2416x; 1.0297x over previous
import functools

import jax
import jax.numpy as jnp
from jax import lax
from jax.experimental import pallas as pl
from jax.experimental.pallas import tpu as pltpu
from jax.experimental.pallas import tpu_sc as plsc

B, D = 2048, 2048
O, F = 2048, 32

NC, NS = 2, 16
NW = NC * NS
OBLK = 32
NBLK = O // OBLK
BLK_PER_W = NBLK // NW
_ZUNROLL = 8


def _sc_scatter_body(idx_hbm, w_hbm, out_hbm, blk, idxs, ws):
    wid = lax.axis_index("s") * NC + lax.axis_index("c")
    zeros16 = jnp.zeros((16,), jnp.float32)

    for rep in range(BLK_PER_W):
        b = wid * BLK_PER_W + rep
        o0 = b * OBLK

        def zero_body(i, carry):
            c = i * 16 * _ZUNROLL
            for u in range(_ZUNROLL):
                blk[pl.ds(c + u * 16, 16)] = zeros16
            return carry

        lax.fori_loop(0, (OBLK * D) // (16 * _ZUNROLL), zero_body, 0)

        pltpu.sync_copy(idx_hbm.at[pl.ds(o0, OBLK)], idxs)
        pltpu.sync_copy(w_hbm.at[pl.ds(o0, OBLK)], ws)

        for ol in range(OBLK):
            base = jnp.full((16,), ol * D, jnp.int32)
            for h in range(F // 16):
                iv = idxs[ol, pl.ds(h * 16, 16)]
                wv = ws[ol, pl.ds(h * 16, 16)]
                plsc.addupdate_scatter(blk, [base + iv], wv)

        pltpu.sync_copy(blk, out_hbm.at[pl.ds(o0 * D, OBLK * D)])


@functools.cache
def _sc_scatter():
    return functools.partial(
        pl.kernel,
        out_type=jax.ShapeDtypeStruct((O * D,), jnp.float32),
        mesh=plsc.VectorSubcoreMesh(
            core_axis_name="c", subcore_axis_name="s", num_cores=NC, num_subcores=NS
        ),
        scratch_types=[
            pltpu.VMEM((OBLK * D,), jnp.float32),
            pltpu.VMEM((OBLK, F), jnp.int32),
            pltpu.VMEM((OBLK, F), jnp.float32),
        ],
        compiler_params=pltpu.CompilerParams(
            use_tc_tiling_on_sc=False, needs_layout_passes=False
        ),
    )(_sc_scatter_body)


TB = 256


def _mm_body(x_ref, w_ref, b_ref, o_ref):
    o_ref[...] = (
        lax.dot_general(
            x_ref[...],
            w_ref[...],
            (((1,), (1,)), ((), ())),
            preferred_element_type=jnp.float32,
        )
        + b_ref[...]
    )


_matmul = pl.pallas_call(
    _mm_body,
    grid=(B // TB,),
    in_specs=[
        pl.BlockSpec((TB, D), lambda i: (i, 0)),
        pl.BlockSpec((O, D), lambda i: (0, 0)),
        pl.BlockSpec((1, O), lambda i: (0, 0)),
    ],
    out_specs=pl.BlockSpec((TB, O), lambda i: (i, 0)),
    out_shape=jax.ShapeDtypeStruct((B, O), jnp.float32),
)


def kernel(input, indx_seqs, weight, bias):
    w_dense = _sc_scatter()(indx_seqs.astype(jnp.int32), weight).reshape(O, D)
    return _matmul(input, w_dense, bias.reshape(1, O))
